# Initial kernel scaffold; baseline (speedup 1.0000x reference)
import jax
import jax.numpy as jnp
from jax.experimental import pallas as pl


def kernel(x, pos_emb):
    raise NotImplementedError("write your pallas kernel here")


# speedup vs baseline: 1.4503x; 1.4503x over previous
import jax
import jax.numpy as jnp
from jax.experimental import pallas as pl


def _add_body(x_ref, p_ref, o_ref):
    o_ref[...] = x_ref[...] + p_ref[...]


def kernel(x, pos_emb):
    B, S, D = x.shape
    LBLK = 512
    grid = (S // LBLK, B)
    return pl.pallas_call(
        _add_body,
        grid=grid,
        in_specs=[
            pl.BlockSpec((1, LBLK, D), lambda i, b: (b, i, 0)),
            pl.BlockSpec((LBLK, D), lambda i, b: (i, 0)),
        ],
        out_specs=pl.BlockSpec((1, LBLK, D), lambda i, b: (b, i, 0)),
        out_shape=jax.ShapeDtypeStruct(x.shape, x.dtype),
    )(x, pos_emb)


# speedup vs baseline: 1.6809x; 1.1590x over previous
import jax
import jax.numpy as jnp
from jax.experimental import pallas as pl


def _add_body(x_ref, p_ref, o_ref):
    o_ref[...] = x_ref[...] + p_ref[...]


def kernel(x, pos_emb):
    B, S, D = x.shape
    LBLK = 1024
    grid = (S // LBLK, B)
    return pl.pallas_call(
        _add_body,
        grid=grid,
        in_specs=[
            pl.BlockSpec((1, LBLK, D), lambda i, b: (b, i, 0)),
            pl.BlockSpec((LBLK, D), lambda i, b: (i, 0)),
        ],
        out_specs=pl.BlockSpec((1, LBLK, D), lambda i, b: (b, i, 0)),
        out_shape=jax.ShapeDtypeStruct(x.shape, x.dtype),
    )(x, pos_emb)


# speedup vs baseline: 1.7963x; 1.0686x over previous
import jax
import jax.numpy as jnp
from jax.experimental import pallas as pl


def _add_body(x_ref, p_ref, o_ref):
    o_ref[...] = x_ref[...] + p_ref[...]


def kernel(x, pos_emb):
    B, S, D = x.shape
    LBLK = 2048
    grid = (S // LBLK, B)
    return pl.pallas_call(
        _add_body,
        grid=grid,
        in_specs=[
            pl.BlockSpec((1, LBLK, D), lambda i, b: (b, i, 0)),
            pl.BlockSpec((LBLK, D), lambda i, b: (i, 0)),
        ],
        out_specs=pl.BlockSpec((1, LBLK, D), lambda i, b: (b, i, 0)),
        out_shape=jax.ShapeDtypeStruct(x.shape, x.dtype),
    )(x, pos_emb)
